# baseline (speedup 1.0000x reference)
import functools

import jax
import jax.numpy as jnp
from jax import lax
from jax.experimental import pallas as pl
from jax.experimental.pallas import tpu as pltpu
from jax.experimental.pallas import tpu_sc as plsc

_NUM_NODES = 10000
_NUM_EDGES = 320000
_NUM_REL = 512
_D = 128
_NW = 32
_EPW = _NUM_EDGES // _NW
_C = 80
_NCHUNK = _EPW // _C
_NBUF = 5
_AHEAD = 3

_mesh = plsc.VectorSubcoreMesh(core_axis_name="c", subcore_axis_name="s")

_GATHER_DNUMS = lax.GatherDimensionNumbers(
    offset_dims=(), collapsed_slice_dims=(0,), start_index_map=(0,))


def _permute(v, idx):
    return lax.gather(v, idx[:, None], _GATHER_DNUMS, slice_sizes=(1,),
                      mode=lax.GatherScatterMode.PROMISE_IN_BOUNDS)


@functools.partial(
    pl.kernel,
    mesh=_mesh,
    compiler_params=pltpu.CompilerParams(needs_layout_passes=False,
                                         use_tc_tiling_on_sc=False),
    out_type=jax.ShapeDtypeStruct((_NUM_EDGES,), jnp.float32),
    scratch_types=[
        pltpu.VMEM((_NUM_REL * _D // 2,), jnp.int32),
        pltpu.VMEM_SHARED((_NUM_NODES, _D // 2), jnp.int32),
        pltpu.VMEM((_NBUF, _C, _D // 2), jnp.int32),
        pltpu.VMEM((_NBUF, _C, _D // 2), jnp.int32),
        pltpu.VMEM((_NBUF, _C), jnp.int32),
        pltpu.VMEM((_NBUF, _C), jnp.int32),
        pltpu.VMEM((_NBUF, _C), jnp.int32),
        pltpu.VMEM((_NBUF, _C), jnp.float32),
    ] + [pltpu.SemaphoreType.DMA] * (3 * _NBUF),
)
def _distmult_sc(z_hbm, src_hbm, dst_hbm, et_hbm, rel_hbm, out_hbm,
                 rel_v, z_sp, srow, drow, sidx, didx, tidx, oc, *sems):
    wid = lax.axis_index("c") * 16 + lax.axis_index("s")
    base = wid * _EPW
    sem_i = sems[:_NBUF]
    sem_r = sems[_NBUF:2 * _NBUF]
    sem_o = sems[2 * _NBUF:]
    @pl.when(lax.axis_index("s") == 0)
    def _():
        pltpu.sync_copy(z_hbm, z_sp)
    pltpu.sync_copy(rel_hbm, rel_v)
    plsc.subcore_barrier()
    lane = lax.iota(jnp.int32, 16)

    def idx_fetch(j, b):
        off = base + j * _C
        pltpu.async_copy(src_hbm.at[pl.ds(off, _C)], sidx.at[b], sem_i[b])
        pltpu.async_copy(dst_hbm.at[pl.ds(off, _C)], didx.at[b], sem_i[b])
        pltpu.async_copy(et_hbm.at[pl.ds(off, _C)], tidx.at[b], sem_i[b])

    def rows_issue(j, b):
        for ref in (sidx, didx, tidx):
            pltpu.make_async_copy(src_hbm.at[pl.ds(base, _C)],
                                  ref.at[b], sem_i[b]).wait()
        h = _C // 2
        pltpu.async_copy(z_sp.at[sidx.at[b, pl.ds(0, h)]],
                         srow.at[b, pl.ds(0, h)], sem_r[b])
        pltpu.async_copy(z_sp.at[sidx.at[b, pl.ds(h, h)]],
                         srow.at[b, pl.ds(h, h)], sem_r[b])
        pltpu.async_copy(z_sp.at[didx.at[b, pl.ds(0, h)]],
                         drow.at[b, pl.ds(0, h)], sem_r[b])
        pltpu.async_copy(z_sp.at[didx.at[b, pl.ds(h, h)]],
                         drow.at[b, pl.ds(h, h)], sem_r[b])

    def process(j, b, first):
        h = _C // 2
        for ref in (srow, srow, drow, drow):
            pltpu.make_async_copy(z_sp.at[sidx.at[b, pl.ds(0, h)]],
                                  ref.at[b, pl.ds(0, h)], sem_r[b]).wait()
        @pl.when(jnp.logical_not(first))
        def _():
            pltpu.make_async_copy(oc.at[b], out_hbm.at[pl.ds(base, _C)],
                                  sem_o[b]).wait()

        def group_body(g, c):
            e0 = g * 16
            tvals = tidx[b, pl.ds(e0, 16)]
            res = jnp.zeros((16,), jnp.float32)
            for jj in range(16):
                e = e0 + jj
                t_spl = _permute(tvals, lane * 0 + jj)
                rbase = t_spl * (_D // 2) + lane
                acc = None
                for k in range(4):
                    sl = plsc.bitcast(srow[b, e, pl.ds(16 * k, 16)],
                                      jnp.bfloat16)
                    dl = plsc.bitcast(drow[b, e, pl.ds(16 * k, 16)],
                                      jnp.bfloat16)
                    rl = plsc.bitcast(
                        plsc.load_gather(rel_v, [rbase + 16 * k]),
                        jnp.bfloat16)
                    s_a, s_b = plsc.unpack(
                        sl, format=plsc.PackFormat.INTERLEAVED)
                    d_a, d_b = plsc.unpack(
                        dl, format=plsc.PackFormat.INTERLEAVED)
                    r_a, r_b = plsc.unpack(
                        rl, format=plsc.PackFormat.INTERLEAVED)
                    p = s_a * d_a * r_a + s_b * d_b * r_b
                    acc = p if acc is None else acc + p
                for m in (8, 4, 2, 1):
                    acc = acc + _permute(acc, lane ^ m)
                res = jnp.where(lane == jj, acc, res)
            oc[b, pl.ds(e0, 16)] = res
            return c

        lax.fori_loop(0, _C // 16, group_body, 0)
        @pl.when(j + _NBUF < _NCHUNK)
        def _():
            idx_fetch(j + _NBUF, b)
        pltpu.async_copy(oc.at[b], out_hbm.at[pl.ds(base + j * _C, _C)],
                         sem_o[b])

    for b in range(_NBUF):
        idx_fetch(b, b)
    for b in range(_AHEAD):
        rows_issue(b, b)

    def block_body(i, carry):
        j0 = _NBUF * i
        for b in range(_NBUF):
            j = j0 + b
            process(j, b, j < _NBUF)

            @pl.when(j + _AHEAD < _NCHUNK)
            def _():
                rows_issue(j + _AHEAD, (b + _AHEAD) % _NBUF)
        return carry

    lax.fori_loop(0, _NCHUNK // _NBUF, block_body, 0)
    for b in range(_NBUF):
        pltpu.make_async_copy(oc.at[b], out_hbm.at[pl.ds(base, _C)],
                              sem_o[b]).wait()


def kernel(z, edge_index, edge_type, rel_emb):
    src = edge_index[0].astype(jnp.int32)
    dst = edge_index[1].astype(jnp.int32)
    et = edge_type.astype(jnp.int32)
    z32 = lax.bitcast_convert_type(
        z.astype(jnp.bfloat16).reshape(_NUM_NODES, _D // 2, 2), jnp.int32)
    rel32 = lax.bitcast_convert_type(
        rel_emb.astype(jnp.bfloat16).reshape(_NUM_REL, _D // 2, 2),
        jnp.int32).reshape(-1)
    return _distmult_sc(z32, src, dst, et, rel32)


# speedup vs baseline: 1.0135x; 1.0135x over previous
import functools

import jax
import jax.numpy as jnp
from jax import lax
from jax.experimental import pallas as pl
from jax.experimental.pallas import tpu as pltpu
from jax.experimental.pallas import tpu_sc as plsc

_NUM_NODES = 10000
_NUM_EDGES = 320000
_NUM_REL = 512
_D = 128
_NW = 32
_EPW = _NUM_EDGES // _NW
_C = 80
_NCHUNK = _EPW // _C
_NBUF = 5
_AHEAD = 3

_mesh = plsc.VectorSubcoreMesh(core_axis_name="c", subcore_axis_name="s")

_GATHER_DNUMS = lax.GatherDimensionNumbers(
    offset_dims=(), collapsed_slice_dims=(0,), start_index_map=(0,))


def _permute(v, idx):
    return lax.gather(v, idx[:, None], _GATHER_DNUMS, slice_sizes=(1,),
                      mode=lax.GatherScatterMode.PROMISE_IN_BOUNDS)


@functools.partial(
    pl.kernel,
    mesh=_mesh,
    compiler_params=pltpu.CompilerParams(needs_layout_passes=False,
                                         use_tc_tiling_on_sc=False),
    out_type=jax.ShapeDtypeStruct((_NUM_EDGES,), jnp.float32),
    scratch_types=[
        pltpu.VMEM((_NUM_REL * _D // 2,), jnp.int32),
        pltpu.VMEM_SHARED((_NUM_NODES, _D // 2), jnp.int32),
        pltpu.VMEM((_NBUF, _C, _D // 2), jnp.int32),
        pltpu.VMEM((_NBUF, _C, _D // 2), jnp.int32),
        pltpu.VMEM((_NBUF, _C), jnp.int32),
        pltpu.VMEM((_NBUF, _C), jnp.int32),
        pltpu.VMEM((_NBUF, _C), jnp.int32),
        pltpu.VMEM((_NBUF, _C), jnp.float32),
    ] + [pltpu.SemaphoreType.DMA] * (3 * _NBUF),
)
def _distmult_sc(z_hbm, src_hbm, dst_hbm, et_hbm, rel_hbm, out_hbm,
                 rel_v, z_sp, srow, drow, sidx, didx, tidx, oc, *sems):
    wid = lax.axis_index("c") * 16 + lax.axis_index("s")
    base = wid * _EPW
    sem_i = sems[:_NBUF]
    sem_r = sems[_NBUF:2 * _NBUF]
    sem_o = sems[2 * _NBUF:]
    @pl.when(lax.axis_index("s") == 0)
    def _():
        pltpu.sync_copy(z_hbm, z_sp)
    pltpu.sync_copy(rel_hbm, rel_v)
    plsc.subcore_barrier()
    lane = lax.iota(jnp.int32, 16)

    def idx_fetch(j, b):
        off = base + j * _C
        pltpu.async_copy(src_hbm.at[pl.ds(off, _C)], sidx.at[b], sem_i[b])
        pltpu.async_copy(dst_hbm.at[pl.ds(off, _C)], didx.at[b], sem_i[b])
        pltpu.async_copy(et_hbm.at[pl.ds(off, _C)], tidx.at[b], sem_i[b])

    def rows_issue(j, b):
        for ref in (sidx, didx, tidx):
            pltpu.make_async_copy(src_hbm.at[pl.ds(base, _C)],
                                  ref.at[b], sem_i[b]).wait()
        pltpu.async_copy(z_sp.at[sidx.at[b]], srow.at[b], sem_r[b])
        pltpu.async_copy(z_sp.at[didx.at[b]], drow.at[b], sem_r[b])

    def process(j, b, first):
        pltpu.make_async_copy(z_sp.at[sidx.at[b]], srow.at[b],
                              sem_r[b]).wait()
        pltpu.make_async_copy(z_sp.at[didx.at[b]], drow.at[b],
                              sem_r[b]).wait()
        @pl.when(jnp.logical_not(first))
        def _():
            pltpu.make_async_copy(oc.at[b], out_hbm.at[pl.ds(base, _C)],
                                  sem_o[b]).wait()

        def group_body(g, c):
            e0 = g * 16
            tvals = tidx[b, pl.ds(e0, 16)]
            res = jnp.zeros((16,), jnp.float32)
            for jj in range(16):
                e = e0 + jj
                t_spl = _permute(tvals, lane * 0 + jj)
                rbase = t_spl * (_D // 2) + lane
                acc = None
                for k in range(4):
                    sl = plsc.bitcast(srow[b, e, pl.ds(16 * k, 16)],
                                      jnp.bfloat16)
                    dl = plsc.bitcast(drow[b, e, pl.ds(16 * k, 16)],
                                      jnp.bfloat16)
                    rl = plsc.bitcast(
                        plsc.load_gather(rel_v, [rbase + 16 * k]),
                        jnp.bfloat16)
                    s_a, s_b = plsc.unpack(
                        sl, format=plsc.PackFormat.INTERLEAVED)
                    d_a, d_b = plsc.unpack(
                        dl, format=plsc.PackFormat.INTERLEAVED)
                    r_a, r_b = plsc.unpack(
                        rl, format=plsc.PackFormat.INTERLEAVED)
                    p = s_a * d_a * r_a + s_b * d_b * r_b
                    acc = p if acc is None else acc + p
                for m in (8, 4, 2, 1):
                    acc = acc + _permute(acc, lane ^ m)
                res = jnp.where(lane == jj, acc, res)
            oc[b, pl.ds(e0, 16)] = res
            return c

        lax.fori_loop(0, _C // 16, group_body, 0)
        @pl.when(j + _NBUF < _NCHUNK)
        def _():
            idx_fetch(j + _NBUF, b)
        pltpu.async_copy(oc.at[b], out_hbm.at[pl.ds(base + j * _C, _C)],
                         sem_o[b])

    for b in range(_NBUF):
        idx_fetch(b, b)
    for b in range(_AHEAD):
        rows_issue(b, b)

    def block_body(i, carry):
        j0 = _NBUF * i
        for b in range(_NBUF):
            j = j0 + b
            process(j, b, j < _NBUF)

            @pl.when(j + _AHEAD < _NCHUNK)
            def _():
                rows_issue(j + _AHEAD, (b + _AHEAD) % _NBUF)
        return carry

    lax.fori_loop(0, _NCHUNK // _NBUF, block_body, 0)
    for b in range(_NBUF):
        pltpu.make_async_copy(oc.at[b], out_hbm.at[pl.ds(base, _C)],
                              sem_o[b]).wait()


def kernel(z, edge_index, edge_type, rel_emb):
    src = edge_index[0].astype(jnp.int32)
    dst = edge_index[1].astype(jnp.int32)
    et = edge_type.astype(jnp.int32)
    z32 = lax.bitcast_convert_type(
        z.astype(jnp.bfloat16).reshape(_NUM_NODES, _D // 2, 2), jnp.int32)
    rel32 = lax.bitcast_convert_type(
        rel_emb.astype(jnp.bfloat16).reshape(_NUM_REL, _D // 2, 2),
        jnp.int32).reshape(-1)
    return _distmult_sc(z32, src, dst, et, rel32)
